# baseline (speedup 1.0000x reference)
import functools

import jax
import jax.numpy as jnp
from jax import lax
from jax.experimental import pallas as pl
from jax.experimental.pallas import tpu as pltpu
from jax.experimental.pallas import tpu_sc as plsc

D_MODEL = 128


def _normalize_table(w):
    v, d = w.shape
    scale = float(d) ** 0.5
    blk = 2000
    assert v % blk == 0

    def body(w_ref, o_ref):
        e = w_ref[...]
        n = jnp.sqrt(jnp.sum(e * e, axis=1, keepdims=True))
        o_ref[...] = e / jnp.maximum(n, 1e-12) * scale

    return pl.pallas_call(
        body,
        grid=(v // blk,),
        in_specs=[pl.BlockSpec((blk, d), lambda i: (i, 0))],
        out_specs=pl.BlockSpec((blk, d), lambda i: (i, 0)),
        out_shape=jax.ShapeDtypeStruct((v, d), jnp.float32),
    )(w)


@functools.lru_cache(maxsize=None)
def _make_sc_gather(n_idx, d):
    info = plsc.get_sparse_core_info()
    nc, ns = info.num_cores, info.num_subcores
    nw = nc * ns
    assert n_idx % nw == 0
    per_w = n_idx // nw
    chunk = 320
    assert per_w % chunk == 0 and chunk % 8 == 0
    n_chunks = per_w // chunk
    assert n_chunks % 2 == 0
    n_groups = n_chunks // 2

    mesh = plsc.VectorSubcoreMesh(core_axis_name="c", subcore_axis_name="s")

    @functools.partial(
        pl.kernel,
        mesh=mesh,
        out_type=jax.ShapeDtypeStruct((n_idx, d), jnp.float32),
        scratch_types=[
            pltpu.VMEM((per_w,), jnp.int32),
            pltpu.VMEM((chunk, d), jnp.float32),
            pltpu.VMEM((chunk, d), jnp.float32),
            pltpu.SemaphoreType.DMA,
            pltpu.SemaphoreType.DMA,
            pltpu.SemaphoreType.DMA,
        ],
    )
    def gather_kernel(
        table_hbm, idx_hbm, out_hbm, idx_v, rows_v0, rows_v1,
        gsem, osem0, osem1,
    ):
        wid = lax.axis_index("s") * nc + lax.axis_index("c")
        base = wid * per_w
        rows_vs = (rows_v0, rows_v1)
        osems = (osem0, osem1)

        pltpu.sync_copy(idx_hbm.at[pl.ds(base, per_w)], idx_v)

        def issue_gather(slot, i):
            pltpu.async_copy(
                table_hbm.at[idx_v.at[pl.ds(i * chunk, chunk)]],
                rows_vs[slot],
                gsem,
            )

        def wait_gather(slot):
            pltpu.make_async_copy(
                table_hbm.at[idx_v.at[pl.ds(0, chunk)]], rows_vs[slot], gsem
            ).wait()

        def issue_out(slot, i):
            off = base + i * chunk
            pltpu.async_copy(
                rows_vs[slot], out_hbm.at[pl.ds(off, chunk)], osems[slot]
            )

        def wait_out(slot):
            pltpu.make_async_copy(
                rows_vs[slot], out_hbm.at[pl.ds(base, chunk)], osems[slot]
            ).wait()

        issue_gather(0, 0)
        issue_gather(1, 1)
        wait_gather(0)
        issue_out(0, 0)
        wait_gather(1)
        issue_out(1, 1)

        def group(g, carry):
            i0 = 2 * g
            for slot in (0, 1):
                i = i0 + slot
                wait_out(slot)
                issue_gather(slot, i)
            for slot in (0, 1):
                wait_gather(slot)
                issue_out(slot, i0 + slot)
            return carry

        lax.fori_loop(1, n_groups, group, 0)
        wait_out(0)
        wait_out(1)

    return gather_kernel


def kernel(x, emb_weight):
    b, t = x.shape
    v, d = emb_weight.shape
    table = _normalize_table(emb_weight)
    idx = x.reshape(-1).astype(jnp.int32)
    out = _make_sc_gather(b * t, d)(table, idx)
    return out.reshape(b, t, d)


# speedup vs baseline: 1.0048x; 1.0048x over previous
import functools

import jax
import jax.numpy as jnp
from jax import lax
from jax.experimental import pallas as pl
from jax.experimental.pallas import tpu as pltpu
from jax.experimental.pallas import tpu_sc as plsc

D_MODEL = 128


def _normalize_table(w):
    v, d = w.shape
    scale = float(d) ** 0.5
    blk = 2000
    assert v % blk == 0

    def body(w_ref, o_ref):
        e = w_ref[...]
        n = jnp.sqrt(jnp.sum(e * e, axis=1, keepdims=True))
        o_ref[...] = e / jnp.maximum(n, 1e-12) * scale

    return pl.pallas_call(
        body,
        grid=(v // blk,),
        in_specs=[pl.BlockSpec((blk, d), lambda i: (i, 0))],
        out_specs=pl.BlockSpec((blk, d), lambda i: (i, 0)),
        out_shape=jax.ShapeDtypeStruct((v, d), jnp.float32),
    )(w)


@functools.lru_cache(maxsize=None)
def _make_sc_gather(n_idx, d):
    info = plsc.get_sparse_core_info()
    nc, ns = info.num_cores, info.num_subcores
    nw = nc * ns
    assert n_idx % nw == 0
    per_w = n_idx // nw
    chunk = 200
    nslots = 4
    assert per_w % chunk == 0 and chunk % 8 == 0
    n_chunks = per_w // chunk
    assert n_chunks % nslots == 0
    n_groups = n_chunks // nslots

    mesh = plsc.VectorSubcoreMesh(core_axis_name="c", subcore_axis_name="s")

    @functools.partial(
        pl.kernel,
        mesh=mesh,
        out_type=jax.ShapeDtypeStruct((n_idx, d), jnp.float32),
        scratch_types=[
            pltpu.VMEM((per_w,), jnp.int32),
        ]
        + [pltpu.VMEM((chunk, d), jnp.float32) for _ in range(nslots)]
        + [pltpu.SemaphoreType.DMA]
        + [pltpu.SemaphoreType.DMA for _ in range(nslots)],
    )
    def gather_kernel(table_hbm, idx_hbm, out_hbm, idx_v, *bufs):
        rows_vs = bufs[:nslots]
        gsem = bufs[nslots]
        osems = bufs[nslots + 1 :]
        wid = lax.axis_index("s") * nc + lax.axis_index("c")
        base = wid * per_w

        pltpu.sync_copy(idx_hbm.at[pl.ds(base, per_w)], idx_v)

        def issue_gather(slot, i):
            pltpu.async_copy(
                table_hbm.at[idx_v.at[pl.ds(i * chunk, chunk)]],
                rows_vs[slot],
                gsem,
            )

        def wait_gather(slot):
            pltpu.make_async_copy(
                table_hbm.at[idx_v.at[pl.ds(0, chunk)]], rows_vs[slot], gsem
            ).wait()

        def issue_out(slot, i):
            off = base + i * chunk
            pltpu.async_copy(
                rows_vs[slot], out_hbm.at[pl.ds(off, chunk)], osems[slot]
            )

        def wait_out(slot):
            pltpu.make_async_copy(
                rows_vs[slot], out_hbm.at[pl.ds(base, chunk)], osems[slot]
            ).wait()

        for slot in range(nslots):
            issue_gather(slot, slot)
        for slot in range(nslots):
            wait_gather(slot)
            issue_out(slot, slot)

        def group(g, carry):
            i0 = nslots * g
            for slot in range(nslots):
                wait_out(slot)
                issue_gather(slot, i0 + slot)
            for slot in range(nslots):
                wait_gather(slot)
                issue_out(slot, i0 + slot)
            return carry

        lax.fori_loop(1, n_groups, group, 0)
        for slot in range(nslots):
            wait_out(slot)

    return gather_kernel


def kernel(x, emb_weight):
    b, t = x.shape
    v, d = emb_weight.shape
    table = _normalize_table(emb_weight)
    idx = x.reshape(-1).astype(jnp.int32)
    out = _make_sc_gather(b * t, d)(table, idx)
    return out.reshape(b, t, d)


# speedup vs baseline: 1.0604x; 1.0553x over previous
import functools

import jax
import jax.numpy as jnp
from jax import lax
from jax.experimental import pallas as pl
from jax.experimental.pallas import tpu as pltpu
from jax.experimental.pallas import tpu_sc as plsc

D_MODEL = 128


def _normalize_table(w):
    v, d = w.shape
    scale = float(d) ** 0.5
    blk = 10000
    assert v % blk == 0

    def body(w_ref, o_ref):
        e = w_ref[...]
        n = jnp.sqrt(jnp.sum(e * e, axis=1, keepdims=True))
        o_ref[...] = e / jnp.maximum(n, 1e-12) * scale

    return pl.pallas_call(
        body,
        grid=(v // blk,),
        in_specs=[pl.BlockSpec((blk, d), lambda i: (i, 0))],
        out_specs=pl.BlockSpec((blk, d), lambda i: (i, 0)),
        out_shape=jax.ShapeDtypeStruct((v, d), jnp.float32),
    )(w)


@functools.lru_cache(maxsize=None)
def _make_sc_gather(b, t, d):
    info = plsc.get_sparse_core_info()
    nc, ns, nl = info.num_cores, info.num_subcores, info.num_lanes
    nw = nc * ns
    assert b % nw == 0
    rows_w = b // nw
    nslots = 2
    assert rows_w % nslots == 0
    n_groups = rows_w // nslots
    n_vec = -(-t // nl)

    mesh = plsc.VectorSubcoreMesh(core_axis_name="c", subcore_axis_name="s")

    @functools.partial(
        pl.kernel,
        mesh=mesh,
        out_type=jax.ShapeDtypeStruct((b, t, d), jnp.float32),
        scratch_types=[
            pltpu.VMEM((rows_w, t), jnp.int32),
            pltpu.VMEM((rows_w * t,), jnp.int32),
        ]
        + [pltpu.VMEM((t, d), jnp.float32) for _ in range(nslots)]
        + [pltpu.SemaphoreType.DMA]
        + [pltpu.SemaphoreType.DMA for _ in range(nslots)],
    )
    def gather_kernel(table_hbm, x_hbm, out_hbm, idx2d_v, idx_v, *bufs):
        rows_vs = bufs[:nslots]
        gsem = bufs[nslots]
        osems = bufs[nslots + 1 :]
        wid = lax.axis_index("s") * nc + lax.axis_index("c")
        base = wid * rows_w

        pltpu.sync_copy(x_hbm.at[pl.ds(base, rows_w)], idx2d_v)

        def repack_row(r, carry):
            for c in range(n_vec):
                col = min(c * nl, t - nl)
                v = idx2d_v[r, pl.ds(col, nl)]
                idx_v[pl.ds(r * t + col, nl)] = v
            return carry

        lax.fori_loop(0, rows_w, repack_row, 0)

        def issue_gather(slot, i):
            pltpu.async_copy(
                table_hbm.at[idx_v.at[pl.ds(i * t, t)]], rows_vs[slot], gsem
            )

        def wait_gather(slot):
            pltpu.make_async_copy(
                table_hbm.at[idx_v.at[pl.ds(0, t)]], rows_vs[slot], gsem
            ).wait()

        def issue_out(slot, i):
            pltpu.async_copy(rows_vs[slot], out_hbm.at[base + i], osems[slot])

        def wait_out(slot):
            pltpu.make_async_copy(
                rows_vs[slot], out_hbm.at[base], osems[slot]
            ).wait()

        for slot in range(nslots):
            issue_gather(slot, slot)
        for slot in range(nslots):
            wait_gather(slot)
            issue_out(slot, slot)

        def group(g, carry):
            i0 = nslots * g
            for slot in range(nslots):
                wait_out(slot)
                issue_gather(slot, i0 + slot)
            for slot in range(nslots):
                wait_gather(slot)
                issue_out(slot, i0 + slot)
            return carry

        lax.fori_loop(1, n_groups, group, 0)
        for slot in range(nslots):
            wait_out(slot)

    return gather_kernel


def kernel(x, emb_weight):
    b, t = x.shape
    v, d = emb_weight.shape
    table = _normalize_table(emb_weight)
    idx = x if x.dtype == jnp.int32 else x.astype(jnp.int32)
    return _make_sc_gather(b, t, d)(table, idx)
